# Initial kernel scaffold; baseline (speedup 1.0000x reference)
import dataclasses
import functools

import jax
import jax.numpy as jnp
from jax import lax
from jax.experimental import pallas as pl
from jax.experimental.pallas import tpu as pltpu
from jax.experimental.pallas import tpu_sc as plsc

N = 10000
E = 320000
EPRIME = E + N
IN = 128
H = 8
C = 16
HC = H * C
NEG = 0.2

NW = 32
B = 128
T = 10368
EP = NW * T
NCH = T // B
NP = 10240
RPT = NP // 16

_mesh = plsc.VectorSubcoreMesh(core_axis_name="c", subcore_axis_name="s")


def _sc_params():
    cp = pltpu.CompilerParams()
    if "needs_layout_passes" in pltpu.CompilerParams.__dataclass_fields__:
        cp = dataclasses.replace(cp, needs_layout_passes=False)
    return cp


def _tc_prep_body(x_ref, w_ref, a1w_ref, a2w_ref, h_ref, a1_ref, a2_ref):
    h = lax.dot_general(x_ref[...], w_ref[...], (((1,), (0,)), ((), ())),
                        precision=lax.Precision.HIGHEST,
                        preferred_element_type=jnp.float32)
    h_ref[...] = h
    a1_ref[...] = lax.dot_general(h, a1w_ref[...], (((1,), (0,)), ((), ())),
                                  precision=lax.Precision.HIGHEST,
                                  preferred_element_type=jnp.float32)
    a2_ref[...] = lax.dot_general(h, a2w_ref[...], (((1,), (0,)), ((), ())),
                                  precision=lax.Precision.HIGHEST,
                                  preferred_element_type=jnp.float32)


def _tc_prep(xp, W, A1, A2):
    blk = 1280
    grid = NP // blk
    return pl.pallas_call(
        _tc_prep_body,
        grid=(grid,),
        in_specs=[
            pl.BlockSpec((blk, IN), lambda i: (i, 0)),
            pl.BlockSpec((IN, HC), lambda i: (0, 0)),
            pl.BlockSpec((IN, 16), lambda i: (0, 0)),
            pl.BlockSpec((IN, 16), lambda i: (0, 0)),
        ],
        out_specs=[
            pl.BlockSpec((blk, HC), lambda i: (i, 0)),
            pl.BlockSpec((blk, 16), lambda i: (i, 0)),
            pl.BlockSpec((blk, 16), lambda i: (i, 0)),
        ],
        out_shape=[
            jax.ShapeDtypeStruct((NP, HC), jnp.float32),
            jax.ShapeDtypeStruct((NP, 16), jnp.float32),
            jax.ShapeDtypeStruct((NP, 16), jnp.float32),
        ],
    )(xp, W, A1, A2)


def _sc1_body(src_hbm, dst_hbm, a1_hbm, a2_hbm, z16_hbm,
              ex_hbm, d0_hbm, d1_hbm,
              idx_s, idx_d, ga1, ga2, den_sh):
    cc = lax.axis_index("c")
    ss = lax.axis_index("s")
    wid = ss * 2 + cc

    pltpu.sync_copy(z16_hbm.at[pl.ds(ss * RPT, RPT)],
                    den_sh.at[pl.ds(ss * RPT, RPT)])
    plsc.subcore_barrier()

    @pl.loop(0, NCH)
    def _chunk(ch):
        base = wid * T + ch * B
        pltpu.sync_copy(src_hbm.at[pl.ds(base, B)], idx_s)
        pltpu.sync_copy(dst_hbm.at[pl.ds(base, B)], idx_d)
        pltpu.sync_copy(a1_hbm.at[idx_s], ga1)
        pltpu.sync_copy(a2_hbm.at[idx_d], ga2)

        @pl.loop(0, B)
        def _row(j):
            v = ga1[pl.ds(j, 1), :] + ga2[pl.ds(j, 1), :]
            v = jnp.where(v >= 0.0, v, NEG * v)
            ga1[pl.ds(j, 1), :] = jnp.exp(v)

        pltpu.sync_copy(ga1, ex_hbm.at[pl.ds(base, B)])
        pltpu.sync_copy(ga1, den_sh.at[idx_d], add=True)

    plsc.subcore_barrier()

    @pl.when(cc == 0)
    def _dump0():
        pltpu.sync_copy(den_sh.at[pl.ds(ss * RPT, RPT)],
                        d0_hbm.at[pl.ds(ss * RPT, RPT)])

    @pl.when(cc == 1)
    def _dump1():
        pltpu.sync_copy(den_sh.at[pl.ds(ss * RPT, RPT)],
                        d1_hbm.at[pl.ds(ss * RPT, RPT)])


def _sc1(src, dst, a1, a2, z16):
    f = pl.kernel(
        _sc1_body,
        out_type=[
            jax.ShapeDtypeStruct((EP, 16), jnp.float32),
            jax.ShapeDtypeStruct((NP, 16), jnp.float32),
            jax.ShapeDtypeStruct((NP, 16), jnp.float32),
        ],
        mesh=_mesh,
        scratch_types=[
            pltpu.VMEM((B,), jnp.int32),
            pltpu.VMEM((B,), jnp.int32),
            pltpu.VMEM((B, 16), jnp.float32),
            pltpu.VMEM((B, 16), jnp.float32),
            pltpu.VMEM_SHARED((NP, 16), jnp.float32),
        ],
        compiler_params=_sc_params(),
    )
    return f(src, dst, a1, a2, z16)


def _sc2_body(src_hbm, dst_hbm, ex_hbm, d0_hbm, d1_hbm, h_hbm, z128_hbm,
              alpha_hbm, o0_hbm, o1_hbm,
              idx_s, idx_d, exb, d0b, d1b, hb, acc_sh):
    cc = lax.axis_index("c")
    ss = lax.axis_index("s")
    wid = ss * 2 + cc

    pltpu.sync_copy(z128_hbm.at[pl.ds(ss * RPT, RPT)],
                    acc_sh.at[pl.ds(ss * RPT, RPT)])
    plsc.subcore_barrier()

    @pl.loop(0, NCH)
    def _chunk(ch):
        base = wid * T + ch * B
        pltpu.sync_copy(src_hbm.at[pl.ds(base, B)], idx_s)
        pltpu.sync_copy(dst_hbm.at[pl.ds(base, B)], idx_d)
        pltpu.sync_copy(ex_hbm.at[pl.ds(base, B)], exb)
        pltpu.sync_copy(d0_hbm.at[idx_d], d0b)
        pltpu.sync_copy(d1_hbm.at[idx_d], d1b)

        @pl.loop(0, B)
        def _norm(j):
            den = d0b[pl.ds(j, 1), :] + d1b[pl.ds(j, 1), :] + 1e-16
            exb[pl.ds(j, 1), :] = exb[pl.ds(j, 1), :] / den

        pltpu.sync_copy(exb, alpha_hbm.at[pl.ds(base, B)])
        pltpu.sync_copy(h_hbm.at[idx_s], hb)

        @pl.loop(0, B)
        def _scale(e):
            ev = lax.broadcast(e, (16,))
            for j in range(H):
                jv = jnp.full((16,), j, jnp.int32)
                spl = plsc.load_gather(exb, [ev, jv])
                hv = hb[pl.ds(e, 1), pl.ds(16 * j, 16)]
                hb[pl.ds(e, 1), pl.ds(16 * j, 16)] = hv * spl.reshape(1, 16)

        pltpu.sync_copy(hb, acc_sh.at[idx_d], add=True)

    plsc.subcore_barrier()

    @pl.when(cc == 0)
    def _dump0():
        pltpu.sync_copy(acc_sh.at[pl.ds(ss * RPT, RPT)],
                        o0_hbm.at[pl.ds(ss * RPT, RPT)])

    @pl.when(cc == 1)
    def _dump1():
        pltpu.sync_copy(acc_sh.at[pl.ds(ss * RPT, RPT)],
                        o1_hbm.at[pl.ds(ss * RPT, RPT)])


def _sc2(src, dst, ex, d0, d1, h, z128):
    f = pl.kernel(
        _sc2_body,
        out_type=[
            jax.ShapeDtypeStruct((EP, 16), jnp.float32),
            jax.ShapeDtypeStruct((NP, HC), jnp.float32),
            jax.ShapeDtypeStruct((NP, HC), jnp.float32),
        ],
        mesh=_mesh,
        scratch_types=[
            pltpu.VMEM((B,), jnp.int32),
            pltpu.VMEM((B,), jnp.int32),
            pltpu.VMEM((B, 16), jnp.float32),
            pltpu.VMEM((B, 16), jnp.float32),
            pltpu.VMEM((B, 16), jnp.float32),
            pltpu.VMEM((B, HC), jnp.float32),
            pltpu.VMEM_SHARED((NP, HC), jnp.float32),
        ],
        compiler_params=_sc_params(),
    )
    return f(src, dst, ex, d0, d1, h, z128)


def _tc_final_body(p0_ref, p1_ref, b_ref, o_ref):
    o_ref[...] = p0_ref[...] + p1_ref[...] + b_ref[...]


def _tc_final(o0, o1, bias2):
    blk = 1000
    grid = N // blk
    return pl.pallas_call(
        _tc_final_body,
        grid=(grid,),
        in_specs=[
            pl.BlockSpec((blk, HC), lambda i: (i, 0)),
            pl.BlockSpec((blk, HC), lambda i: (i, 0)),
            pl.BlockSpec((1, HC), lambda i: (0, 0)),
        ],
        out_specs=pl.BlockSpec((blk, HC), lambda i: (i, 0)),
        out_shape=jax.ShapeDtypeStruct((N, HC), jnp.float32),
    )(o0, o1, bias2)


def kernel(x, edge_index, W, att_src, att_dst, bias):
    n = x.shape[0]
    idt = edge_index.dtype
    loop = jnp.arange(n, dtype=idt)
    ei = jnp.concatenate([edge_index, jnp.stack([loop, loop], axis=0)], axis=1)

    src = jnp.concatenate(
        [ei[0].astype(jnp.int32), jnp.zeros((EP - EPRIME,), jnp.int32)])
    dst = jnp.concatenate(
        [ei[1].astype(jnp.int32), jnp.full((EP - EPRIME,), N, jnp.int32)])

    xp = jnp.pad(x, ((0, NP - n), (0, 0)))
    eye = jnp.eye(H, dtype=jnp.float32)
    A_src = (att_src.reshape(H, C)[:, :, None] * eye[:, None, :]).reshape(HC, H)
    A_dst = (att_dst.reshape(H, C)[:, :, None] * eye[:, None, :]).reshape(HC, H)
    A1 = jnp.concatenate([A_src, A_dst], axis=1)
    A2 = jnp.concatenate([A_dst, A_src], axis=1)

    h, a1, a2 = _tc_prep(xp, W, A1, A2)

    z16 = jnp.zeros((NP, 16), jnp.float32)
    z128 = jnp.zeros((NP, HC), jnp.float32)

    ex, d0, d1 = _sc1(src, dst, a1, a2, z16)
    alpha, o0, o1 = _sc2(src, dst, ex, d0, d1, h, z128)

    out = _tc_final(o0, o1, bias.reshape(1, HC))
    alpha_n = alpha[:EPRIME, :H]
    return out, (ei, alpha_n)


# speedup vs baseline: 25.9069x; 25.9069x over previous
import dataclasses
import functools

import jax
import jax.numpy as jnp
from jax import lax
from jax.experimental import pallas as pl
from jax.experimental.pallas import tpu as pltpu
from jax.experimental.pallas import tpu_sc as plsc

N = 10000
E = 320000
EPRIME = E + N
IN = 128
H = 8
C = 16
HC = H * C
NEG = 0.2

NW = 32
B = 128
T = 10368
EP = NW * T
NCH = T // B
NP = 10240
RPT = NP // 16

_mesh = plsc.VectorSubcoreMesh(core_axis_name="c", subcore_axis_name="s")


def _sc_params():
    cp = pltpu.CompilerParams()
    fields = pltpu.CompilerParams.__dataclass_fields__
    if "needs_layout_passes" in fields:
        cp = dataclasses.replace(cp, needs_layout_passes=False)
    if "use_tc_tiling_on_sc" in fields:
        cp = dataclasses.replace(cp, use_tc_tiling_on_sc=False)
    return cp


def _tc_prep_body(x_ref, w_ref, a1w_ref, a2w_ref, h_ref, a1_ref, a2_ref):
    h = lax.dot_general(x_ref[...], w_ref[...], (((1,), (0,)), ((), ())),
                        precision=lax.Precision.HIGHEST,
                        preferred_element_type=jnp.float32)
    h_ref[...] = h
    a1_ref[...] = lax.dot_general(h, a1w_ref[...], (((1,), (0,)), ((), ())),
                                  precision=lax.Precision.HIGHEST,
                                  preferred_element_type=jnp.float32)
    a2_ref[...] = lax.dot_general(h, a2w_ref[...], (((1,), (0,)), ((), ())),
                                  precision=lax.Precision.HIGHEST,
                                  preferred_element_type=jnp.float32)


def _tc_prep(xp, W, A1, A2):
    blk = 1280
    grid = NP // blk
    return pl.pallas_call(
        _tc_prep_body,
        grid=(grid,),
        in_specs=[
            pl.BlockSpec((blk, IN), lambda i: (i, 0)),
            pl.BlockSpec((IN, HC), lambda i: (0, 0)),
            pl.BlockSpec((IN, 16), lambda i: (0, 0)),
            pl.BlockSpec((IN, 16), lambda i: (0, 0)),
        ],
        out_specs=[
            pl.BlockSpec((blk, HC), lambda i: (i, 0)),
            pl.BlockSpec((blk, 16), lambda i: (i, 0)),
            pl.BlockSpec((blk, 16), lambda i: (i, 0)),
        ],
        out_shape=[
            jax.ShapeDtypeStruct((NP, HC), jnp.float32),
            jax.ShapeDtypeStruct((NP, 16), jnp.float32),
            jax.ShapeDtypeStruct((NP, 16), jnp.float32),
        ],
    )(xp, W, A1, A2)


def _sc1_body(src_hbm, dst_hbm, a1_hbm, a2_hbm, z16_hbm,
              ex_hbm, d0_hbm, d1_hbm,
              idx_s, idx_d, ga1, ga2, den_sh):
    cc = lax.axis_index("c")
    ss = lax.axis_index("s")
    wid = ss * 2 + cc

    pltpu.sync_copy(z16_hbm.at[pl.ds(ss * RPT, RPT)],
                    den_sh.at[pl.ds(ss * RPT, RPT)])
    plsc.subcore_barrier()

    @pl.loop(0, NCH)
    def _chunk(ch):
        base = wid * T + ch * B
        pltpu.sync_copy(src_hbm.at[pl.ds(base, B)], idx_s)
        pltpu.sync_copy(dst_hbm.at[pl.ds(base, B)], idx_d)
        pltpu.sync_copy(a1_hbm.at[idx_s], ga1)
        pltpu.sync_copy(a2_hbm.at[idx_d], ga2)

        @pl.loop(0, B)
        def _row(j):
            v = ga1[j, :] + ga2[j, :]
            v = jnp.where(v >= 0.0, v, NEG * v)
            ga1[j, :] = jnp.exp(v)

        pltpu.sync_copy(ga1, ex_hbm.at[pl.ds(base, B)])
        pltpu.sync_copy(ga1, den_sh.at[idx_d], add=True)

    plsc.subcore_barrier()

    @pl.when(cc == 0)
    def _dump0():
        pltpu.sync_copy(den_sh.at[pl.ds(ss * RPT, RPT)],
                        d0_hbm.at[pl.ds(ss * RPT, RPT)])

    @pl.when(cc == 1)
    def _dump1():
        pltpu.sync_copy(den_sh.at[pl.ds(ss * RPT, RPT)],
                        d1_hbm.at[pl.ds(ss * RPT, RPT)])


def _sc1(src, dst, a1, a2, z16):
    f = pl.kernel(
        _sc1_body,
        out_type=[
            jax.ShapeDtypeStruct((EP, 16), jnp.float32),
            jax.ShapeDtypeStruct((NP, 16), jnp.float32),
            jax.ShapeDtypeStruct((NP, 16), jnp.float32),
        ],
        mesh=_mesh,
        scratch_types=[
            pltpu.VMEM((B,), jnp.int32),
            pltpu.VMEM((B,), jnp.int32),
            pltpu.VMEM((B, 16), jnp.float32),
            pltpu.VMEM((B, 16), jnp.float32),
            pltpu.VMEM_SHARED((NP, 16), jnp.float32),
        ],
        compiler_params=_sc_params(),
    )
    return f(src, dst, a1, a2, z16)


def _sc2_body(src_hbm, dst_hbm, ex_hbm, d0_hbm, d1_hbm, h_hbm, z128_hbm,
              alpha_hbm, o0_hbm, o1_hbm,
              idx_s, idx_d, exb, d0b, d1b, hb, acc_sh):
    cc = lax.axis_index("c")
    ss = lax.axis_index("s")
    wid = ss * 2 + cc

    pltpu.sync_copy(z128_hbm.at[pl.ds(ss * RPT, RPT)],
                    acc_sh.at[pl.ds(ss * RPT, RPT)])
    plsc.subcore_barrier()

    @pl.loop(0, NCH)
    def _chunk(ch):
        base = wid * T + ch * B
        pltpu.sync_copy(src_hbm.at[pl.ds(base, B)], idx_s)
        pltpu.sync_copy(dst_hbm.at[pl.ds(base, B)], idx_d)
        pltpu.sync_copy(ex_hbm.at[pl.ds(base, B)], exb)
        pltpu.sync_copy(d0_hbm.at[idx_d], d0b)
        pltpu.sync_copy(d1_hbm.at[idx_d], d1b)

        @pl.loop(0, B)
        def _norm(j):
            den = d0b[j, :] + d1b[j, :] + 1e-16
            exb[j, :] = exb[j, :] / den

        pltpu.sync_copy(exb, alpha_hbm.at[pl.ds(base, B)])
        pltpu.sync_copy(h_hbm.at[idx_s], hb)

        @pl.loop(0, B)
        def _scale(e):
            ev = lax.broadcast(e, (16,))
            for j in range(H):
                jv = jnp.full((16,), j, jnp.int32)
                spl = plsc.load_gather(exb, [ev, jv])
                hv = hb[e, pl.ds(16 * j, 16)]
                hb[e, pl.ds(16 * j, 16)] = hv * spl

        pltpu.sync_copy(hb, acc_sh.at[idx_d], add=True)

    plsc.subcore_barrier()

    @pl.when(cc == 0)
    def _dump0():
        pltpu.sync_copy(acc_sh.at[pl.ds(ss * RPT, RPT)],
                        o0_hbm.at[pl.ds(ss * RPT, RPT)])

    @pl.when(cc == 1)
    def _dump1():
        pltpu.sync_copy(acc_sh.at[pl.ds(ss * RPT, RPT)],
                        o1_hbm.at[pl.ds(ss * RPT, RPT)])


def _sc2(src, dst, ex, d0, d1, h, z128):
    f = pl.kernel(
        _sc2_body,
        out_type=[
            jax.ShapeDtypeStruct((EP, 16), jnp.float32),
            jax.ShapeDtypeStruct((NP, HC), jnp.float32),
            jax.ShapeDtypeStruct((NP, HC), jnp.float32),
        ],
        mesh=_mesh,
        scratch_types=[
            pltpu.VMEM((B,), jnp.int32),
            pltpu.VMEM((B,), jnp.int32),
            pltpu.VMEM((B, 16), jnp.float32),
            pltpu.VMEM((B, 16), jnp.float32),
            pltpu.VMEM((B, 16), jnp.float32),
            pltpu.VMEM((B, HC), jnp.float32),
            pltpu.VMEM_SHARED((NP, HC), jnp.float32),
        ],
        compiler_params=_sc_params(),
    )
    return f(src, dst, ex, d0, d1, h, z128)


def _tc_final_body(p0_ref, p1_ref, b_ref, o_ref):
    o_ref[...] = p0_ref[...] + p1_ref[...] + b_ref[...]


def _tc_final(o0, o1, bias2):
    blk = 1000
    grid = N // blk
    return pl.pallas_call(
        _tc_final_body,
        grid=(grid,),
        in_specs=[
            pl.BlockSpec((blk, HC), lambda i: (i, 0)),
            pl.BlockSpec((blk, HC), lambda i: (i, 0)),
            pl.BlockSpec((1, HC), lambda i: (0, 0)),
        ],
        out_specs=pl.BlockSpec((blk, HC), lambda i: (i, 0)),
        out_shape=jax.ShapeDtypeStruct((N, HC), jnp.float32),
    )(o0, o1, bias2)


def kernel(x, edge_index, W, att_src, att_dst, bias):
    n = x.shape[0]
    idt = edge_index.dtype
    loop = jnp.arange(n, dtype=idt)
    ei = jnp.concatenate([edge_index, jnp.stack([loop, loop], axis=0)], axis=1)

    src = jnp.concatenate(
        [ei[0].astype(jnp.int32), jnp.zeros((EP - EPRIME,), jnp.int32)])
    dst = jnp.concatenate(
        [ei[1].astype(jnp.int32), jnp.full((EP - EPRIME,), N, jnp.int32)])

    xp = jnp.pad(x, ((0, NP - n), (0, 0)))
    eye = jnp.eye(H, dtype=jnp.float32)
    A_src = (att_src.reshape(H, C)[:, :, None] * eye[:, None, :]).reshape(HC, H)
    A_dst = (att_dst.reshape(H, C)[:, :, None] * eye[:, None, :]).reshape(HC, H)
    A1 = jnp.concatenate([A_src, A_dst], axis=1)
    A2 = jnp.concatenate([A_dst, A_src], axis=1)

    h, a1, a2 = _tc_prep(xp, W, A1, A2)

    z16 = jnp.zeros((NP, 16), jnp.float32)
    z128 = jnp.zeros((NP, HC), jnp.float32)

    ex, d0, d1 = _sc1(src, dst, a1, a2, z16)
    alpha, o0, o1 = _sc2(src, dst, ex, d0, d1, h, z128)

    out = _tc_final(o0, o1, bias.reshape(1, HC))
    alpha_n = alpha[:EPRIME, :H]
    return out, (ei, alpha_n)


# speedup vs baseline: 42.4004x; 1.6366x over previous
import dataclasses

import jax
import jax.numpy as jnp
from jax import lax
from jax.experimental import pallas as pl
from jax.experimental.pallas import tpu as pltpu
from jax.experimental.pallas import tpu_sc as plsc

N = 10000
E = 320000
EPRIME = E + N
IN = 128
H = 8
C = 16
HC = H * C
NEG = 0.2

NW = 32
B = 128
NCH = 82
T = NCH * B
EP = NW * T
NP = 10240
RPT = NP // 16

_mesh = plsc.VectorSubcoreMesh(core_axis_name="c", subcore_axis_name="s")

_GDN = lax.GatherDimensionNumbers(
    offset_dims=(), collapsed_slice_dims=(0,), start_index_map=(0,))


def _splat(vec, j):
    idx = jnp.full((16, 1), j, dtype=jnp.int32)
    return lax.gather(vec, idx, _GDN, (1,),
                      mode=lax.GatherScatterMode.PROMISE_IN_BOUNDS)


def _sc_params():
    cp = pltpu.CompilerParams()
    fields = pltpu.CompilerParams.__dataclass_fields__
    if "needs_layout_passes" in fields:
        cp = dataclasses.replace(cp, needs_layout_passes=False)
    if "use_tc_tiling_on_sc" in fields:
        cp = dataclasses.replace(cp, use_tc_tiling_on_sc=False)
    return cp


def _tc_prep_body(x_ref, w_ref, a1w_ref, a2w_ref, h_ref, a1_ref, a2_ref):
    h = lax.dot_general(x_ref[...], w_ref[...], (((1,), (0,)), ((), ())),
                        precision=lax.Precision.HIGHEST,
                        preferred_element_type=jnp.float32)
    h_ref[...] = h
    a1_ref[...] = lax.dot_general(h, a1w_ref[...], (((1,), (0,)), ((), ())),
                                  precision=lax.Precision.HIGHEST,
                                  preferred_element_type=jnp.float32)
    a2_ref[...] = lax.dot_general(h, a2w_ref[...], (((1,), (0,)), ((), ())),
                                  precision=lax.Precision.HIGHEST,
                                  preferred_element_type=jnp.float32)


def _tc_prep(xp, W, A1, A2):
    blk = 1280
    grid = NP // blk
    return pl.pallas_call(
        _tc_prep_body,
        grid=(grid,),
        in_specs=[
            pl.BlockSpec((blk, IN), lambda i: (i, 0)),
            pl.BlockSpec((IN, HC), lambda i: (0, 0)),
            pl.BlockSpec((IN, 16), lambda i: (0, 0)),
            pl.BlockSpec((IN, 16), lambda i: (0, 0)),
        ],
        out_specs=[
            pl.BlockSpec((blk, HC), lambda i: (i, 0)),
            pl.BlockSpec((blk, 16), lambda i: (i, 0)),
            pl.BlockSpec((blk, 16), lambda i: (i, 0)),
        ],
        out_shape=[
            jax.ShapeDtypeStruct((NP, HC), jnp.float32),
            jax.ShapeDtypeStruct((NP, 16), jnp.float32),
            jax.ShapeDtypeStruct((NP, 16), jnp.float32),
        ],
    )(xp, W, A1, A2)


def _copy_idx_whole(src2d, ch, dst1d):
    for k in range(B // 16):
        dst1d[pl.ds(16 * k, 16)] = src2d[ch, pl.ds(16 * k, 16)]


def _sc1_body(src_hbm, dst_hbm, a1_hbm, a2_hbm, z16_hbm,
              ex_hbm, d0_hbm, d1_hbm,
              srcM, dstM, didxs_0, didxs_1,
              ga1_0, ga1_1, ga2_0, ga2_1, den_sh,
              gsem0, gsem1):
    cc = lax.axis_index("c")
    ss = lax.axis_index("s")
    wid = ss * 2 + cc
    tbase = wid * T

    ga1 = (ga1_0, ga1_1)
    ga2 = (ga2_0, ga2_1)
    didxs = (didxs_0, didxs_1)
    gsem = (gsem0, gsem1)

    pltpu.sync_copy(z16_hbm.at[pl.ds(ss * RPT, RPT)],
                    den_sh.at[pl.ds(ss * RPT, RPT)])
    pltpu.sync_copy(src_hbm.at[wid], srcM)
    pltpu.sync_copy(dst_hbm.at[wid], dstM)
    plsc.subcore_barrier()

    def fire_gathers(ch, p):
        pltpu.async_copy(a1_hbm.at[srcM.at[ch]], ga1[p], gsem[p])
        pltpu.async_copy(a2_hbm.at[dstM.at[ch]], ga2[p], gsem[p])

    def drain_gathers(p):
        pltpu.make_async_copy(a1_hbm.at[pl.ds(0, B)], ga1[p], gsem[p]).wait()
        pltpu.make_async_copy(a2_hbm.at[pl.ds(0, B)], ga2[p], gsem[p]).wait()

    fire_gathers(0, 0)

    @pl.loop(0, NCH // 2)
    def _outer(it):
        for p in (0, 1):
            ch = 2 * it + p
            drain_gathers(p)
            _copy_idx_whole(dstM, ch, didxs[p])

            @pl.when(ch + 1 < NCH)
            def _prefetch():
                fire_gathers(ch + 1, 1 - p)

            @pl.loop(0, B)
            def _row(j):
                v = ga1[p][j, :] + ga2[p][j, :]
                v = jnp.where(v >= 0.0, v, NEG * v)
                ga1[p][j, :] = jnp.exp(v)

            pltpu.sync_copy(ga1[p], ex_hbm.at[pl.ds(tbase + ch * B, B)])
            pltpu.sync_copy(ga1[p], den_sh.at[didxs[p]], add=True)

    plsc.subcore_barrier()

    @pl.when(cc == 0)
    def _dump0():
        pltpu.sync_copy(den_sh.at[pl.ds(ss * RPT, RPT)],
                        d0_hbm.at[pl.ds(ss * RPT, RPT)])

    @pl.when(cc == 1)
    def _dump1():
        pltpu.sync_copy(den_sh.at[pl.ds(ss * RPT, RPT)],
                        d1_hbm.at[pl.ds(ss * RPT, RPT)])


def _sc1(src3, dst3, a1, a2, z16):
    f = pl.kernel(
        _sc1_body,
        out_type=[
            jax.ShapeDtypeStruct((EP, 16), jnp.float32),
            jax.ShapeDtypeStruct((NP, 16), jnp.float32),
            jax.ShapeDtypeStruct((NP, 16), jnp.float32),
        ],
        mesh=_mesh,
        scratch_types=[
            pltpu.VMEM((NCH, B), jnp.int32),
            pltpu.VMEM((NCH, B), jnp.int32),
            pltpu.VMEM((B,), jnp.int32),
            pltpu.VMEM((B,), jnp.int32),
            pltpu.VMEM((B, 16), jnp.float32),
            pltpu.VMEM((B, 16), jnp.float32),
            pltpu.VMEM((B, 16), jnp.float32),
            pltpu.VMEM((B, 16), jnp.float32),
            pltpu.VMEM_SHARED((NP, 16), jnp.float32),
            pltpu.SemaphoreType.DMA,
            pltpu.SemaphoreType.DMA,
        ],
        compiler_params=_sc_params(),
    )
    return f(src3, dst3, a1, a2, z16)


def _sc2_body(src_hbm, dst_hbm, ex_hbm, d0_hbm, d1_hbm, h_hbm, z128_hbm,
              alpha_hbm, o0_hbm, o1_hbm,
              sidx_0, sidx_1, didxg_0, didxg_1, didxs_0, didxs_1,
              exb_0, exb_1, d0b_0, d0b_1, d1b_0, d1b_1,
              hb_0, hb_1, acc_sh,
              gsem0, gsem1, isem0, isem1):
    cc = lax.axis_index("c")
    ss = lax.axis_index("s")
    wid = ss * 2 + cc
    tbase = wid * T

    sidx = (sidx_0, sidx_1)
    didxg = (didxg_0, didxg_1)
    didxs = (didxs_0, didxs_1)
    exb = (exb_0, exb_1)
    d0b = (d0b_0, d0b_1)
    d1b = (d1b_0, d1b_1)
    hb = (hb_0, hb_1)
    gsem = (gsem0, gsem1)
    isem = (isem0, isem1)

    pltpu.sync_copy(z128_hbm.at[pl.ds(ss * RPT, RPT)],
                    acc_sh.at[pl.ds(ss * RPT, RPT)])
    plsc.subcore_barrier()

    def fire_idx(ch, q):
        pltpu.async_copy(src_hbm.at[wid, ch], sidx[q], isem[q])
        pltpu.async_copy(dst_hbm.at[wid, ch], didxg[q], isem[q])

    def drain_idx(q):
        pltpu.make_async_copy(src_hbm.at[0, 0], sidx[q], isem[q]).wait()
        pltpu.make_async_copy(dst_hbm.at[0, 0], didxg[q], isem[q]).wait()

    def fire_gathers(ch, p):
        pltpu.async_copy(ex_hbm.at[pl.ds(tbase + ch * B, B)], exb[p], gsem[p])
        pltpu.async_copy(d0_hbm.at[didxg[p]], d0b[p], gsem[p])
        pltpu.async_copy(d1_hbm.at[didxg[p]], d1b[p], gsem[p])
        pltpu.async_copy(h_hbm.at[sidx[p]], hb[p], gsem[p])

    def drain_gathers(p):
        pltpu.make_async_copy(ex_hbm.at[pl.ds(0, B)], exb[p], gsem[p]).wait()
        pltpu.make_async_copy(d0_hbm.at[pl.ds(0, B)], d0b[p], gsem[p]).wait()
        pltpu.make_async_copy(d1_hbm.at[pl.ds(0, B)], d1b[p], gsem[p]).wait()
        pltpu.make_async_copy(h_hbm.at[pl.ds(0, B)], hb[p], gsem[p]).wait()

    pltpu.sync_copy(src_hbm.at[wid, 0], sidx[0])
    pltpu.sync_copy(dst_hbm.at[wid, 0], didxg[0])
    fire_idx(1, 1)
    fire_gathers(0, 0)

    @pl.loop(0, NCH // 2)
    def _outer(it):
        for p in (0, 1):
            ch = 2 * it + p
            drain_gathers(p)
            _copy_idx_whole_1d(didxg[p], didxs[p])

            @pl.when(ch + 1 < NCH)
            def _prefetch():
                drain_idx(1 - p)
                fire_gathers(ch + 1, 1 - p)

                @pl.when(ch + 2 < NCH)
                def _nextidx():
                    fire_idx(ch + 2, p)

            @pl.loop(0, B)
            def _edge(j):
                den = d0b[p][j, :] + d1b[p][j, :] + 1e-16
                anv = exb[p][j, :] / den
                exb[p][j, :] = anv
                for hh in range(H):
                    spl = _splat(anv, hh)
                    hv = hb[p][j, pl.ds(16 * hh, 16)]
                    hb[p][j, pl.ds(16 * hh, 16)] = hv * spl

            pltpu.sync_copy(exb[p], alpha_hbm.at[pl.ds(tbase + ch * B, B)])
            pltpu.sync_copy(hb[p], acc_sh.at[didxs[p]], add=True)

    plsc.subcore_barrier()

    @pl.when(cc == 0)
    def _dump0():
        pltpu.sync_copy(acc_sh.at[pl.ds(ss * RPT, RPT)],
                        o0_hbm.at[pl.ds(ss * RPT, RPT)])

    @pl.when(cc == 1)
    def _dump1():
        pltpu.sync_copy(acc_sh.at[pl.ds(ss * RPT, RPT)],
                        o1_hbm.at[pl.ds(ss * RPT, RPT)])


def _copy_idx_whole_1d(src1d, dst1d):
    for k in range(B // 16):
        dst1d[pl.ds(16 * k, 16)] = src1d[pl.ds(16 * k, 16)]


def _sc2(src3, dst3, ex, d0, d1, h, z128):
    f = pl.kernel(
        _sc2_body,
        out_type=[
            jax.ShapeDtypeStruct((EP, 16), jnp.float32),
            jax.ShapeDtypeStruct((NP, HC), jnp.float32),
            jax.ShapeDtypeStruct((NP, HC), jnp.float32),
        ],
        mesh=_mesh,
        scratch_types=[
            pltpu.VMEM((B,), jnp.int32),
            pltpu.VMEM((B,), jnp.int32),
            pltpu.VMEM((B,), jnp.int32),
            pltpu.VMEM((B,), jnp.int32),
            pltpu.VMEM((B,), jnp.int32),
            pltpu.VMEM((B,), jnp.int32),
            pltpu.VMEM((B, 16), jnp.float32),
            pltpu.VMEM((B, 16), jnp.float32),
            pltpu.VMEM((B, 16), jnp.float32),
            pltpu.VMEM((B, 16), jnp.float32),
            pltpu.VMEM((B, 16), jnp.float32),
            pltpu.VMEM((B, 16), jnp.float32),
            pltpu.VMEM((B, HC), jnp.float32),
            pltpu.VMEM((B, HC), jnp.float32),
            pltpu.VMEM_SHARED((NP, HC), jnp.float32),
            pltpu.SemaphoreType.DMA,
            pltpu.SemaphoreType.DMA,
            pltpu.SemaphoreType.DMA,
            pltpu.SemaphoreType.DMA,
        ],
        compiler_params=_sc_params(),
    )
    return f(src3, dst3, ex, d0, d1, h, z128)


def _tc_final_body(p0_ref, p1_ref, b_ref, o_ref):
    o_ref[...] = p0_ref[...] + p1_ref[...] + b_ref[...]


def _tc_final(o0, o1, bias2):
    blk = 1000
    grid = N // blk
    return pl.pallas_call(
        _tc_final_body,
        grid=(grid,),
        in_specs=[
            pl.BlockSpec((blk, HC), lambda i: (i, 0)),
            pl.BlockSpec((blk, HC), lambda i: (i, 0)),
            pl.BlockSpec((1, HC), lambda i: (0, 0)),
        ],
        out_specs=pl.BlockSpec((blk, HC), lambda i: (i, 0)),
        out_shape=jax.ShapeDtypeStruct((N, HC), jnp.float32),
    )(o0, o1, bias2)


def kernel(x, edge_index, W, att_src, att_dst, bias):
    n = x.shape[0]
    idt = edge_index.dtype
    loop = jnp.arange(n, dtype=idt)
    ei = jnp.concatenate([edge_index, jnp.stack([loop, loop], axis=0)], axis=1)

    src = jnp.concatenate(
        [ei[0].astype(jnp.int32), jnp.zeros((EP - EPRIME,), jnp.int32)])
    dst = jnp.concatenate(
        [ei[1].astype(jnp.int32), jnp.full((EP - EPRIME,), N, jnp.int32)])
    src3 = src.reshape(NW, NCH, B)
    dst3 = dst.reshape(NW, NCH, B)

    xp = jnp.pad(x, ((0, NP - n), (0, 0)))
    eye = jnp.eye(H, dtype=jnp.float32)
    A_src = (att_src.reshape(H, C)[:, :, None] * eye[:, None, :]).reshape(HC, H)
    A_dst = (att_dst.reshape(H, C)[:, :, None] * eye[:, None, :]).reshape(HC, H)
    A1 = jnp.concatenate([A_src, A_dst], axis=1)
    A2 = jnp.concatenate([A_dst, A_src], axis=1)

    h, a1, a2 = _tc_prep(xp, W, A1, A2)

    z16 = jnp.zeros((NP, 16), jnp.float32)
    z128 = jnp.zeros((NP, HC), jnp.float32)

    ex, d0, d1 = _sc1(src3, dst3, a1, a2, z16)
    alpha, o0, o1 = _sc2(src3, dst3, ex, d0, d1, h, z128)

    out = _tc_final(o0, o1, bias.reshape(1, HC))
    alpha_n = alpha[:EPRIME, :H]
    return out, (ei, alpha_n)


# speedup vs baseline: 43.0781x; 1.0160x over previous
import dataclasses

import jax
import jax.numpy as jnp
from jax import lax
from jax.experimental import pallas as pl
from jax.experimental.pallas import tpu as pltpu
from jax.experimental.pallas import tpu_sc as plsc

N = 10000
E = 320000
EPRIME = E + N
IN = 128
H = 8
C = 16
HC = H * C
NEG = 0.2

NW = 32
B = 128
NCH = 82
T = NCH * B
EP = NW * T
NP = 10240
RPT = NP // 16

_mesh = plsc.VectorSubcoreMesh(core_axis_name="c", subcore_axis_name="s")

_GDN = lax.GatherDimensionNumbers(
    offset_dims=(), collapsed_slice_dims=(0,), start_index_map=(0,))


def _splat(vec, j):
    idx = jnp.full((16, 1), j, dtype=jnp.int32)
    return lax.gather(vec, idx, _GDN, (1,),
                      mode=lax.GatherScatterMode.PROMISE_IN_BOUNDS)


def _sc_params():
    cp = pltpu.CompilerParams()
    fields = pltpu.CompilerParams.__dataclass_fields__
    if "needs_layout_passes" in fields:
        cp = dataclasses.replace(cp, needs_layout_passes=False)
    if "use_tc_tiling_on_sc" in fields:
        cp = dataclasses.replace(cp, use_tc_tiling_on_sc=False)
    return cp


def _tc_prep_body(x_ref, w_ref, a1w_ref, a2w_ref, h_ref, a1_ref, a2_ref):
    h = lax.dot_general(x_ref[...], w_ref[...], (((1,), (0,)), ((), ())),
                        precision=lax.Precision.HIGHEST,
                        preferred_element_type=jnp.float32)
    h_ref[...] = h
    a1_ref[...] = lax.dot_general(h, a1w_ref[...], (((1,), (0,)), ((), ())),
                                  precision=lax.Precision.HIGHEST,
                                  preferred_element_type=jnp.float32)
    a2_ref[...] = lax.dot_general(h, a2w_ref[...], (((1,), (0,)), ((), ())),
                                  precision=lax.Precision.HIGHEST,
                                  preferred_element_type=jnp.float32)


def _tc_prep(xp, W, A1, A2):
    blk = 1280
    grid = NP // blk
    return pl.pallas_call(
        _tc_prep_body,
        grid=(grid,),
        in_specs=[
            pl.BlockSpec((blk, IN), lambda i: (i, 0)),
            pl.BlockSpec((IN, HC), lambda i: (0, 0)),
            pl.BlockSpec((IN, 16), lambda i: (0, 0)),
            pl.BlockSpec((IN, 16), lambda i: (0, 0)),
        ],
        out_specs=[
            pl.BlockSpec((blk, HC), lambda i: (i, 0)),
            pl.BlockSpec((blk, 16), lambda i: (i, 0)),
            pl.BlockSpec((blk, 16), lambda i: (i, 0)),
        ],
        out_shape=[
            jax.ShapeDtypeStruct((NP, HC), jnp.float32),
            jax.ShapeDtypeStruct((NP, 16), jnp.float32),
            jax.ShapeDtypeStruct((NP, 16), jnp.float32),
        ],
    )(xp, W, A1, A2)


def _copy_idx_whole(src2d, ch, dst1d):
    for k in range(B // 16):
        dst1d[pl.ds(16 * k, 16)] = src2d[ch, pl.ds(16 * k, 16)]


def _sc1_body(src_hbm, dst_hbm, a1_hbm, a2_hbm,
              ex_hbm, d0_hbm, d1_hbm,
              srcM, dstM, didxs_0, didxs_1,
              ga1_0, ga1_1, ga2_0, ga2_1, den_sh,
              gsem0, gsem1):
    cc = lax.axis_index("c")
    ss = lax.axis_index("s")
    wid = ss * 2 + cc
    tbase = wid * T

    ga1 = (ga1_0, ga1_1)
    ga2 = (ga2_0, ga2_1)
    didxs = (didxs_0, didxs_1)
    gsem = (gsem0, gsem1)

    @pl.loop(0, B)
    def _z(j):
        ga1_0[j, :] = jnp.zeros((16,), jnp.float32)

    for r in range(RPT // B):
        pltpu.sync_copy(ga1_0, den_sh.at[pl.ds(ss * RPT + r * B, B)])
    pltpu.sync_copy(src_hbm.at[wid], srcM)
    pltpu.sync_copy(dst_hbm.at[wid], dstM)
    plsc.subcore_barrier()

    def fire_gathers(ch, p):
        pltpu.async_copy(a1_hbm.at[srcM.at[ch]], ga1[p], gsem[p])
        pltpu.async_copy(a2_hbm.at[dstM.at[ch]], ga2[p], gsem[p])

    def drain_gathers(p):
        pltpu.make_async_copy(a1_hbm.at[pl.ds(0, B)], ga1[p], gsem[p]).wait()
        pltpu.make_async_copy(a2_hbm.at[pl.ds(0, B)], ga2[p], gsem[p]).wait()

    fire_gathers(0, 0)

    @pl.loop(0, NCH // 2)
    def _outer(it):
        for p in (0, 1):
            ch = 2 * it + p
            drain_gathers(p)
            _copy_idx_whole(dstM, ch, didxs[p])

            @pl.when(ch + 1 < NCH)
            def _prefetch():
                fire_gathers(ch + 1, 1 - p)

            @pl.loop(0, B)
            def _row(j):
                v = ga1[p][j, :] + ga2[p][j, :]
                v = jnp.where(v >= 0.0, v, NEG * v)
                ga1[p][j, :] = jnp.exp(v)

            pltpu.sync_copy(ga1[p], ex_hbm.at[pl.ds(tbase + ch * B, B)])
            pltpu.sync_copy(ga1[p], den_sh.at[didxs[p]], add=True)

    plsc.subcore_barrier()

    @pl.when(cc == 0)
    def _dump0():
        pltpu.sync_copy(den_sh.at[pl.ds(ss * RPT, RPT)],
                        d0_hbm.at[pl.ds(ss * RPT, RPT)])

    @pl.when(cc == 1)
    def _dump1():
        pltpu.sync_copy(den_sh.at[pl.ds(ss * RPT, RPT)],
                        d1_hbm.at[pl.ds(ss * RPT, RPT)])


def _sc1(src3, dst3, a1, a2):
    f = pl.kernel(
        _sc1_body,
        out_type=[
            jax.ShapeDtypeStruct((EP, 16), jnp.float32),
            jax.ShapeDtypeStruct((NP, 16), jnp.float32),
            jax.ShapeDtypeStruct((NP, 16), jnp.float32),
        ],
        mesh=_mesh,
        scratch_types=[
            pltpu.VMEM((NCH, B), jnp.int32),
            pltpu.VMEM((NCH, B), jnp.int32),
            pltpu.VMEM((B,), jnp.int32),
            pltpu.VMEM((B,), jnp.int32),
            pltpu.VMEM((B, 16), jnp.float32),
            pltpu.VMEM((B, 16), jnp.float32),
            pltpu.VMEM((B, 16), jnp.float32),
            pltpu.VMEM((B, 16), jnp.float32),
            pltpu.VMEM_SHARED((NP, 16), jnp.float32),
            pltpu.SemaphoreType.DMA,
            pltpu.SemaphoreType.DMA,
        ],
        compiler_params=_sc_params(),
    )
    return f(src3, dst3, a1, a2)


def _sc2_body(src_hbm, dst_hbm, ex_hbm, d0_hbm, d1_hbm, h_hbm,
              alpha_hbm, o0_hbm, o1_hbm,
              sidx_0, sidx_1, didxg_0, didxg_1, didxs_0, didxs_1,
              exb_0, exb_1, d0b_0, d0b_1, d1b_0, d1b_1,
              hb_0, hb_1, acc_sh,
              gsem0, gsem1, isem0, isem1):
    cc = lax.axis_index("c")
    ss = lax.axis_index("s")
    wid = ss * 2 + cc
    tbase = wid * T

    sidx = (sidx_0, sidx_1)
    didxg = (didxg_0, didxg_1)
    didxs = (didxs_0, didxs_1)
    exb = (exb_0, exb_1)
    d0b = (d0b_0, d0b_1)
    d1b = (d1b_0, d1b_1)
    hb = (hb_0, hb_1)
    gsem = (gsem0, gsem1)
    isem = (isem0, isem1)

    @pl.loop(0, B)
    def _z(j):
        for k in range(HC // 16):
            hb_0[j, pl.ds(16 * k, 16)] = jnp.zeros((16,), jnp.float32)

    for r in range(RPT // B):
        pltpu.sync_copy(hb_0, acc_sh.at[pl.ds(ss * RPT + r * B, B)])
    plsc.subcore_barrier()

    def fire_idx(ch, q):
        pltpu.async_copy(src_hbm.at[wid, ch], sidx[q], isem[q])
        pltpu.async_copy(dst_hbm.at[wid, ch], didxg[q], isem[q])

    def drain_idx(q):
        pltpu.make_async_copy(src_hbm.at[0, 0], sidx[q], isem[q]).wait()
        pltpu.make_async_copy(dst_hbm.at[0, 0], didxg[q], isem[q]).wait()

    def fire_gathers(ch, p):
        pltpu.async_copy(ex_hbm.at[pl.ds(tbase + ch * B, B)], exb[p], gsem[p])
        pltpu.async_copy(d0_hbm.at[didxg[p]], d0b[p], gsem[p])
        pltpu.async_copy(d1_hbm.at[didxg[p]], d1b[p], gsem[p])
        pltpu.async_copy(h_hbm.at[sidx[p]], hb[p], gsem[p])

    def drain_gathers(p):
        pltpu.make_async_copy(ex_hbm.at[pl.ds(0, B)], exb[p], gsem[p]).wait()
        pltpu.make_async_copy(d0_hbm.at[pl.ds(0, B)], d0b[p], gsem[p]).wait()
        pltpu.make_async_copy(d1_hbm.at[pl.ds(0, B)], d1b[p], gsem[p]).wait()
        pltpu.make_async_copy(h_hbm.at[pl.ds(0, B)], hb[p], gsem[p]).wait()

    pltpu.sync_copy(src_hbm.at[wid, 0], sidx[0])
    pltpu.sync_copy(dst_hbm.at[wid, 0], didxg[0])
    fire_idx(1, 1)
    fire_gathers(0, 0)

    @pl.loop(0, NCH // 2)
    def _outer(it):
        for p in (0, 1):
            ch = 2 * it + p
            drain_gathers(p)
            _copy_idx_whole_1d(didxg[p], didxs[p])

            @pl.when(ch + 1 < NCH)
            def _prefetch():
                drain_idx(1 - p)
                fire_gathers(ch + 1, 1 - p)

                @pl.when(ch + 2 < NCH)
                def _nextidx():
                    fire_idx(ch + 2, p)

            @pl.loop(0, B)
            def _edge(j):
                den = d0b[p][j, :] + d1b[p][j, :] + 1e-16
                anv = exb[p][j, :] / den
                exb[p][j, :] = anv
                for hh in range(H):
                    spl = _splat(anv, hh)
                    hv = hb[p][j, pl.ds(16 * hh, 16)]
                    hb[p][j, pl.ds(16 * hh, 16)] = hv * spl

            pltpu.sync_copy(exb[p], alpha_hbm.at[pl.ds(tbase + ch * B, B)])
            pltpu.sync_copy(hb[p], acc_sh.at[didxs[p]], add=True)

    plsc.subcore_barrier()

    @pl.when(cc == 0)
    def _dump0():
        pltpu.sync_copy(acc_sh.at[pl.ds(ss * RPT, RPT)],
                        o0_hbm.at[pl.ds(ss * RPT, RPT)])

    @pl.when(cc == 1)
    def _dump1():
        pltpu.sync_copy(acc_sh.at[pl.ds(ss * RPT, RPT)],
                        o1_hbm.at[pl.ds(ss * RPT, RPT)])


def _copy_idx_whole_1d(src1d, dst1d):
    for k in range(B // 16):
        dst1d[pl.ds(16 * k, 16)] = src1d[pl.ds(16 * k, 16)]


def _sc2(src3, dst3, ex, d0, d1, h):
    f = pl.kernel(
        _sc2_body,
        out_type=[
            jax.ShapeDtypeStruct((EP, 16), jnp.float32),
            jax.ShapeDtypeStruct((NP, HC), jnp.float32),
            jax.ShapeDtypeStruct((NP, HC), jnp.float32),
        ],
        mesh=_mesh,
        scratch_types=[
            pltpu.VMEM((B,), jnp.int32),
            pltpu.VMEM((B,), jnp.int32),
            pltpu.VMEM((B,), jnp.int32),
            pltpu.VMEM((B,), jnp.int32),
            pltpu.VMEM((B,), jnp.int32),
            pltpu.VMEM((B,), jnp.int32),
            pltpu.VMEM((B, 16), jnp.float32),
            pltpu.VMEM((B, 16), jnp.float32),
            pltpu.VMEM((B, 16), jnp.float32),
            pltpu.VMEM((B, 16), jnp.float32),
            pltpu.VMEM((B, 16), jnp.float32),
            pltpu.VMEM((B, 16), jnp.float32),
            pltpu.VMEM((B, HC), jnp.float32),
            pltpu.VMEM((B, HC), jnp.float32),
            pltpu.VMEM_SHARED((NP, HC), jnp.float32),
            pltpu.SemaphoreType.DMA,
            pltpu.SemaphoreType.DMA,
            pltpu.SemaphoreType.DMA,
            pltpu.SemaphoreType.DMA,
        ],
        compiler_params=_sc_params(),
    )
    return f(src3, dst3, ex, d0, d1, h)


def _tc_final_body(p0_ref, p1_ref, b_ref, o_ref):
    o_ref[...] = p0_ref[...] + p1_ref[...] + b_ref[...]


def _tc_final(o0, o1, bias2):
    blk = 1000
    grid = N // blk
    return pl.pallas_call(
        _tc_final_body,
        grid=(grid,),
        in_specs=[
            pl.BlockSpec((blk, HC), lambda i: (i, 0)),
            pl.BlockSpec((blk, HC), lambda i: (i, 0)),
            pl.BlockSpec((1, HC), lambda i: (0, 0)),
        ],
        out_specs=pl.BlockSpec((blk, HC), lambda i: (i, 0)),
        out_shape=jax.ShapeDtypeStruct((N, HC), jnp.float32),
    )(o0, o1, bias2)


def kernel(x, edge_index, W, att_src, att_dst, bias):
    n = x.shape[0]
    idt = edge_index.dtype
    loop = jnp.arange(n, dtype=idt)
    ei = jnp.concatenate([edge_index, jnp.stack([loop, loop], axis=0)], axis=1)

    src = jnp.concatenate(
        [ei[0].astype(jnp.int32), jnp.zeros((EP - EPRIME,), jnp.int32)])
    dum = N + (jnp.arange(EP - EPRIME, dtype=jnp.int32) % (NP - N))
    dst = jnp.concatenate([ei[1].astype(jnp.int32), dum])
    src3 = src.reshape(NW, NCH, B)
    dst3 = dst.reshape(NW, NCH, B)

    xp = jnp.pad(x, ((0, NP - n), (0, 0)))
    eye = jnp.eye(H, dtype=jnp.float32)
    A_src = (att_src.reshape(H, C)[:, :, None] * eye[:, None, :]).reshape(HC, H)
    A_dst = (att_dst.reshape(H, C)[:, :, None] * eye[:, None, :]).reshape(HC, H)
    A1 = jnp.concatenate([A_src, A_dst], axis=1)
    A2 = jnp.concatenate([A_dst, A_src], axis=1)

    h, a1, a2 = _tc_prep(xp, W, A1, A2)

    ex, d0, d1 = _sc1(src3, dst3, a1, a2)
    alpha, o0, o1 = _sc2(src3, dst3, ex, d0, d1, h)

    out = _tc_final(o0, o1, bias.reshape(1, HC))
    alpha_n = alpha[:EPRIME, :H]
    return out, (ei, alpha_n)


# speedup vs baseline: 43.4640x; 1.0090x over previous
import dataclasses

import jax
import jax.numpy as jnp
from jax import lax
from jax.experimental import pallas as pl
from jax.experimental.pallas import tpu as pltpu
from jax.experimental.pallas import tpu_sc as plsc

N = 10000
E = 320000
EPRIME = E + N
IN = 128
H = 8
C = 16
HC = H * C
NEG = 0.2

NW = 32
B = 128
NCH = 82
T = NCH * B
EP = NW * T
NP = 10240
RPT = NP // 16

_mesh = plsc.VectorSubcoreMesh(core_axis_name="c", subcore_axis_name="s")

_GDN = lax.GatherDimensionNumbers(
    offset_dims=(), collapsed_slice_dims=(0,), start_index_map=(0,))


def _splat(vec, j):
    idx = jnp.full((16, 1), j, dtype=jnp.int32)
    return lax.gather(vec, idx, _GDN, (1,),
                      mode=lax.GatherScatterMode.PROMISE_IN_BOUNDS)


def _sc_params():
    cp = pltpu.CompilerParams()
    fields = pltpu.CompilerParams.__dataclass_fields__
    if "needs_layout_passes" in fields:
        cp = dataclasses.replace(cp, needs_layout_passes=False)
    if "use_tc_tiling_on_sc" in fields:
        cp = dataclasses.replace(cp, use_tc_tiling_on_sc=False)
    return cp


def _tc_prep_body(x_ref, w_ref, a1w_ref, a2w_ref, h_ref, a1_ref, a2_ref):
    h = lax.dot_general(x_ref[...], w_ref[...], (((1,), (0,)), ((), ())),
                        precision=lax.Precision.HIGHEST,
                        preferred_element_type=jnp.float32)
    h_ref[...] = h
    a1_ref[...] = lax.dot_general(h, a1w_ref[...], (((1,), (0,)), ((), ())),
                                  precision=lax.Precision.HIGHEST,
                                  preferred_element_type=jnp.float32)
    a2_ref[...] = lax.dot_general(h, a2w_ref[...], (((1,), (0,)), ((), ())),
                                  precision=lax.Precision.HIGHEST,
                                  preferred_element_type=jnp.float32)


def _tc_prep(xp, W, A1, A2):
    blk = 1280
    grid = NP // blk
    return pl.pallas_call(
        _tc_prep_body,
        grid=(grid,),
        in_specs=[
            pl.BlockSpec((blk, IN), lambda i: (i, 0)),
            pl.BlockSpec((IN, HC), lambda i: (0, 0)),
            pl.BlockSpec((IN, 16), lambda i: (0, 0)),
            pl.BlockSpec((IN, 16), lambda i: (0, 0)),
        ],
        out_specs=[
            pl.BlockSpec((blk, HC), lambda i: (i, 0)),
            pl.BlockSpec((blk, 16), lambda i: (i, 0)),
            pl.BlockSpec((blk, 16), lambda i: (i, 0)),
        ],
        out_shape=[
            jax.ShapeDtypeStruct((NP, HC), jnp.float32),
            jax.ShapeDtypeStruct((NP, 16), jnp.float32),
            jax.ShapeDtypeStruct((NP, 16), jnp.float32),
        ],
    )(xp, W, A1, A2)


def _copy_idx_whole(src2d, ch, dst1d):
    for k in range(B // 16):
        dst1d[pl.ds(16 * k, 16)] = src2d[ch, pl.ds(16 * k, 16)]


def _sc1_body(src_hbm, dst_hbm, a1_hbm, a2_hbm,
              ex_hbm, d0_hbm, d1_hbm,
              srcM, dstM, didxs_0, didxs_1,
              ga1_0, ga1_1, ga2_0, ga2_1, den_sh,
              gsem0, gsem1, ssem0, ssem1):
    cc = lax.axis_index("c")
    ss = lax.axis_index("s")
    wid = ss * 2 + cc
    tbase = wid * T

    ga1 = (ga1_0, ga1_1)
    ga2 = (ga2_0, ga2_1)
    didxs = (didxs_0, didxs_1)
    gsem = (gsem0, gsem1)
    ssem = (ssem0, ssem1)

    @pl.loop(0, B)
    def _z(j):
        ga1_0[j, :] = jnp.zeros((16,), jnp.float32)

    for r in range(RPT // B):
        pltpu.sync_copy(ga1_0, den_sh.at[pl.ds(ss * RPT + r * B, B)])
    pltpu.sync_copy(src_hbm.at[wid], srcM)
    pltpu.sync_copy(dst_hbm.at[wid], dstM)
    plsc.subcore_barrier()

    def fire_gathers(ch, p):
        pltpu.async_copy(a1_hbm.at[srcM.at[ch]], ga1[p], gsem[p])
        pltpu.async_copy(a2_hbm.at[dstM.at[ch]], ga2[p], gsem[p])

    def drain_gathers(p):
        pltpu.make_async_copy(a1_hbm.at[pl.ds(0, B)], ga1[p], gsem[p]).wait()
        pltpu.make_async_copy(a2_hbm.at[pl.ds(0, B)], ga2[p], gsem[p]).wait()

    def drain_stores(q):
        pltpu.make_async_copy(ga1[q], ex_hbm.at[pl.ds(0, B)],
                              ssem[q]).wait()

    fire_gathers(0, 0)

    @pl.loop(0, NCH // 2)
    def _outer(it):
        for p in (0, 1):
            ch = 2 * it + p
            drain_gathers(p)
            _copy_idx_whole(dstM, ch, didxs[p])

            @pl.when(ch + 1 < NCH)
            def _prefetch():
                @pl.when(ch >= 1)
                def _reclaim():
                    drain_stores(1 - p)
                fire_gathers(ch + 1, 1 - p)

            @pl.loop(0, B)
            def _row(j):
                v = ga1[p][j, :] + ga2[p][j, :]
                v = jnp.where(v >= 0.0, v, NEG * v)
                ga1[p][j, :] = jnp.exp(v)

            pltpu.async_copy(ga1[p], ex_hbm.at[pl.ds(tbase + ch * B, B)],
                             ssem[p])
            pltpu.sync_copy(ga1[p], den_sh.at[didxs[p]], add=True)

    drain_stores(0)
    drain_stores(1)
    plsc.subcore_barrier()

    @pl.when(cc == 0)
    def _dump0():
        pltpu.sync_copy(den_sh.at[pl.ds(ss * RPT, RPT)],
                        d0_hbm.at[pl.ds(ss * RPT, RPT)])

    @pl.when(cc == 1)
    def _dump1():
        pltpu.sync_copy(den_sh.at[pl.ds(ss * RPT, RPT)],
                        d1_hbm.at[pl.ds(ss * RPT, RPT)])


def _sc1(src3, dst3, a1, a2):
    f = pl.kernel(
        _sc1_body,
        out_type=[
            jax.ShapeDtypeStruct((EP, 16), jnp.float32),
            jax.ShapeDtypeStruct((NP, 16), jnp.float32),
            jax.ShapeDtypeStruct((NP, 16), jnp.float32),
        ],
        mesh=_mesh,
        scratch_types=[
            pltpu.VMEM((NCH, B), jnp.int32),
            pltpu.VMEM((NCH, B), jnp.int32),
            pltpu.VMEM((B,), jnp.int32),
            pltpu.VMEM((B,), jnp.int32),
            pltpu.VMEM((B, 16), jnp.float32),
            pltpu.VMEM((B, 16), jnp.float32),
            pltpu.VMEM((B, 16), jnp.float32),
            pltpu.VMEM((B, 16), jnp.float32),
            pltpu.VMEM_SHARED((NP, 16), jnp.float32),
            pltpu.SemaphoreType.DMA,
            pltpu.SemaphoreType.DMA,
            pltpu.SemaphoreType.DMA,
            pltpu.SemaphoreType.DMA,
        ],
        compiler_params=_sc_params(),
    )
    return f(src3, dst3, a1, a2)


def _sc2_body(src_hbm, dst_hbm, ex_hbm, d0_hbm, d1_hbm, h_hbm,
              alpha_hbm, o0_hbm, o1_hbm,
              sidx_0, sidx_1, didxg_0, didxg_1, didxs_0, didxs_1,
              exb_0, exb_1, d0b_0, d0b_1, d1b_0, d1b_1,
              hb_0, hb_1, acc_sh,
              gsem0, gsem1, isem0, isem1, ssem0, ssem1):
    cc = lax.axis_index("c")
    ss = lax.axis_index("s")
    wid = ss * 2 + cc
    tbase = wid * T

    sidx = (sidx_0, sidx_1)
    didxg = (didxg_0, didxg_1)
    didxs = (didxs_0, didxs_1)
    exb = (exb_0, exb_1)
    d0b = (d0b_0, d0b_1)
    d1b = (d1b_0, d1b_1)
    hb = (hb_0, hb_1)
    gsem = (gsem0, gsem1)
    isem = (isem0, isem1)
    ssem = (ssem0, ssem1)

    @pl.loop(0, B)
    def _z(j):
        for k in range(HC // 16):
            hb_0[j, pl.ds(16 * k, 16)] = jnp.zeros((16,), jnp.float32)

    for r in range(RPT // B):
        pltpu.sync_copy(hb_0, acc_sh.at[pl.ds(ss * RPT + r * B, B)])
    plsc.subcore_barrier()

    def fire_idx(ch, q):
        pltpu.async_copy(src_hbm.at[wid, ch], sidx[q], isem[q])
        pltpu.async_copy(dst_hbm.at[wid, ch], didxg[q], isem[q])

    def drain_idx(q):
        pltpu.make_async_copy(src_hbm.at[0, 0], sidx[q], isem[q]).wait()
        pltpu.make_async_copy(dst_hbm.at[0, 0], didxg[q], isem[q]).wait()

    def fire_gathers(ch, p):
        pltpu.async_copy(ex_hbm.at[pl.ds(tbase + ch * B, B)], exb[p], gsem[p])
        pltpu.async_copy(d0_hbm.at[didxg[p]], d0b[p], gsem[p])
        pltpu.async_copy(d1_hbm.at[didxg[p]], d1b[p], gsem[p])
        pltpu.async_copy(h_hbm.at[sidx[p]], hb[p], gsem[p])

    def drain_gathers(p):
        pltpu.make_async_copy(ex_hbm.at[pl.ds(0, B)], exb[p], gsem[p]).wait()
        pltpu.make_async_copy(d0_hbm.at[pl.ds(0, B)], d0b[p], gsem[p]).wait()
        pltpu.make_async_copy(d1_hbm.at[pl.ds(0, B)], d1b[p], gsem[p]).wait()
        pltpu.make_async_copy(h_hbm.at[pl.ds(0, B)], hb[p], gsem[p]).wait()

    def drain_stores(q):
        pltpu.make_async_copy(exb[q], alpha_hbm.at[pl.ds(0, B)],
                              ssem[q]).wait()

    pltpu.sync_copy(src_hbm.at[wid, 0], sidx[0])
    pltpu.sync_copy(dst_hbm.at[wid, 0], didxg[0])
    fire_idx(1, 1)
    fire_gathers(0, 0)

    @pl.loop(0, NCH // 2)
    def _outer(it):
        for p in (0, 1):
            ch = 2 * it + p
            drain_gathers(p)
            _copy_idx_whole_1d(didxg[p], didxs[p])

            @pl.when(ch + 1 < NCH)
            def _prefetch():
                @pl.when(ch >= 1)
                def _reclaim():
                    drain_stores(1 - p)
                drain_idx(1 - p)
                fire_gathers(ch + 1, 1 - p)

                @pl.when(ch + 2 < NCH)
                def _nextidx():
                    fire_idx(ch + 2, p)

            @pl.loop(0, B)
            def _edge(j):
                den = d0b[p][j, :] + d1b[p][j, :] + 1e-16
                anv = exb[p][j, :] / den
                exb[p][j, :] = anv
                for hh in range(H):
                    spl = _splat(anv, hh)
                    hv = hb[p][j, pl.ds(16 * hh, 16)]
                    hb[p][j, pl.ds(16 * hh, 16)] = hv * spl

            pltpu.async_copy(exb[p], alpha_hbm.at[pl.ds(tbase + ch * B, B)],
                             ssem[p])
            pltpu.sync_copy(hb[p], acc_sh.at[didxs[p]], add=True)

    drain_stores(0)
    drain_stores(1)
    plsc.subcore_barrier()

    @pl.when(cc == 0)
    def _dump0():
        pltpu.sync_copy(acc_sh.at[pl.ds(ss * RPT, RPT)],
                        o0_hbm.at[pl.ds(ss * RPT, RPT)])

    @pl.when(cc == 1)
    def _dump1():
        pltpu.sync_copy(acc_sh.at[pl.ds(ss * RPT, RPT)],
                        o1_hbm.at[pl.ds(ss * RPT, RPT)])


def _copy_idx_whole_1d(src1d, dst1d):
    for k in range(B // 16):
        dst1d[pl.ds(16 * k, 16)] = src1d[pl.ds(16 * k, 16)]


def _sc2(src3, dst3, ex, d0, d1, h):
    f = pl.kernel(
        _sc2_body,
        out_type=[
            jax.ShapeDtypeStruct((EP, 16), jnp.float32),
            jax.ShapeDtypeStruct((NP, HC), jnp.float32),
            jax.ShapeDtypeStruct((NP, HC), jnp.float32),
        ],
        mesh=_mesh,
        scratch_types=[
            pltpu.VMEM((B,), jnp.int32),
            pltpu.VMEM((B,), jnp.int32),
            pltpu.VMEM((B,), jnp.int32),
            pltpu.VMEM((B,), jnp.int32),
            pltpu.VMEM((B,), jnp.int32),
            pltpu.VMEM((B,), jnp.int32),
            pltpu.VMEM((B, 16), jnp.float32),
            pltpu.VMEM((B, 16), jnp.float32),
            pltpu.VMEM((B, 16), jnp.float32),
            pltpu.VMEM((B, 16), jnp.float32),
            pltpu.VMEM((B, 16), jnp.float32),
            pltpu.VMEM((B, 16), jnp.float32),
            pltpu.VMEM((B, HC), jnp.float32),
            pltpu.VMEM((B, HC), jnp.float32),
            pltpu.VMEM_SHARED((NP, HC), jnp.float32),
            pltpu.SemaphoreType.DMA,
            pltpu.SemaphoreType.DMA,
            pltpu.SemaphoreType.DMA,
            pltpu.SemaphoreType.DMA,
            pltpu.SemaphoreType.DMA,
            pltpu.SemaphoreType.DMA,
        ],
        compiler_params=_sc_params(),
    )
    return f(src3, dst3, ex, d0, d1, h)


def _tc_final_body(p0_ref, p1_ref, b_ref, o_ref):
    o_ref[...] = p0_ref[...] + p1_ref[...] + b_ref[...]


def _tc_final(o0, o1, bias2):
    blk = 1000
    grid = N // blk
    return pl.pallas_call(
        _tc_final_body,
        grid=(grid,),
        in_specs=[
            pl.BlockSpec((blk, HC), lambda i: (i, 0)),
            pl.BlockSpec((blk, HC), lambda i: (i, 0)),
            pl.BlockSpec((1, HC), lambda i: (0, 0)),
        ],
        out_specs=pl.BlockSpec((blk, HC), lambda i: (i, 0)),
        out_shape=jax.ShapeDtypeStruct((N, HC), jnp.float32),
    )(o0, o1, bias2)


def kernel(x, edge_index, W, att_src, att_dst, bias):
    n = x.shape[0]
    idt = edge_index.dtype
    loop = jnp.arange(n, dtype=idt)
    ei = jnp.concatenate([edge_index, jnp.stack([loop, loop], axis=0)], axis=1)

    src = jnp.concatenate(
        [ei[0].astype(jnp.int32), jnp.zeros((EP - EPRIME,), jnp.int32)])
    dum = N + (jnp.arange(EP - EPRIME, dtype=jnp.int32) % (NP - N))
    dst = jnp.concatenate([ei[1].astype(jnp.int32), dum])
    src3 = src.reshape(NW, NCH, B)
    dst3 = dst.reshape(NW, NCH, B)

    xp = jnp.pad(x, ((0, NP - n), (0, 0)))
    eye = jnp.eye(H, dtype=jnp.float32)
    A_src = (att_src.reshape(H, C)[:, :, None] * eye[:, None, :]).reshape(HC, H)
    A_dst = (att_dst.reshape(H, C)[:, :, None] * eye[:, None, :]).reshape(HC, H)
    A1 = jnp.concatenate([A_src, A_dst], axis=1)
    A2 = jnp.concatenate([A_dst, A_src], axis=1)

    h, a1, a2 = _tc_prep(xp, W, A1, A2)

    ex, d0, d1 = _sc1(src3, dst3, a1, a2)
    alpha, o0, o1 = _sc2(src3, dst3, ex, d0, d1, h)

    out = _tc_final(o0, o1, bias.reshape(1, HC))
    alpha_n = alpha[:EPRIME, :H]
    return out, (ei, alpha_n)
